# baseline (speedup 1.0000x reference)
import functools

import jax
import jax.numpy as jnp
from jax import lax
from jax.experimental import pallas as pl
from jax.experimental.pallas import tpu as pltpu
from jax.experimental.pallas import tpu_sc as plsc

B, L, V, D = 16384, 64, 1000000, 8
BL = B * L
H = L * D
NSLAB = B // 8


_NC, _NS = 2, 16
_NW = _NC * _NS
_PER_W = BL // _NW
_CH = 2048
_NCH = _PER_W // _CH

def _sc_gather(table, idx, n_rows):
    mesh = plsc.VectorSubcoreMesh(core_axis_name="c", subcore_axis_name="s")
    per_w = n_rows // _NW
    nch = per_w // _CH

    @functools.partial(
        pl.kernel,
        mesh=mesh,
        compiler_params=pltpu.CompilerParams(use_tc_tiling_on_sc=False),
        out_type=jax.ShapeDtypeStruct((n_rows, D), jnp.float32),
        scratch_types=[
            pltpu.VMEM((_CH,), jnp.int32),
            pltpu.VMEM((_CH, D), jnp.float32),
            pltpu.SemaphoreType.DMA,
        ],
    )
    def gather_kernel(idx_hbm, table_hbm, out_hbm, idx_v, rows_v, sem):
        wid = lax.axis_index("s") * _NC + lax.axis_index("c")
        base = wid * per_w

        def body(i, carry):
            start = base + i * _CH
            pltpu.sync_copy(idx_hbm.at[pl.ds(start, _CH)], idx_v)
            pltpu.async_copy(table_hbm.at[idx_v], rows_v, sem).wait()
            pltpu.sync_copy(rows_v, out_hbm.at[pl.ds(start, _CH)])
            return carry

        lax.fori_loop(0, nch, body, 0)

    return gather_kernel(idx, table)


_BB = 512


def _mlp(x, emb4, E, W1, b1, W2, b2, W3, b3, W4, b4, W5, b5, nb):
    bf16 = jnp.bfloat16

    def dotb(a, w_ref):
        return jnp.dot(a.astype(bf16), w_ref[...],
                       preferred_element_type=jnp.float32)

    def mlp_kernel(x_ref, emb_ref, E_ref, W1_ref, b1_ref, W2_ref, b2_ref,
                   W3_ref, b3_ref, W4_ref, b4_ref, W5_ref, b5_ref, out_ref):
        m = (x_ref[...] != 0).astype(bf16)
        mask = jnp.dot(m, E_ref[...],
                       preferred_element_type=jnp.float32)
        e4 = emb_ref[...].reshape(_BB // 8, 4, 8, 128)
        h = None
        for ct in range(4):
            e = e4[:, ct, :, :].reshape(_BB, 128)
            e = e * mask[:, 128 * ct:128 * (ct + 1)]
            part = jnp.dot(e.astype(bf16),
                           W1_ref[pl.ds(128 * ct, 128), :],
                           preferred_element_type=jnp.float32)
            h = part if h is None else h + part
        h = jnp.maximum(h + b1_ref[...], 0.0)
        h = jnp.maximum(dotb(h, W2_ref) + b2_ref[...], 0.0)
        h = jnp.maximum(dotb(h, W3_ref) + b3_ref[...], 0.0)
        h = jnp.maximum(dotb(h, W4_ref) + b4_ref[...], 0.0)
        out_ref[...] = dotb(h, W5_ref) + b5_ref[...]

    grid = (nb,)
    full = lambda shape: pl.BlockSpec(shape, lambda i: tuple(0 for _ in shape))
    return pl.pallas_call(
        mlp_kernel,
        grid=grid,
        in_specs=[
            pl.BlockSpec((_BB, L), lambda i: (i, 0)),
            pl.BlockSpec((_BB * H // 128, 128), lambda i: (i, 0)),
            full((L, H)),
            full((512, 512)), full((1, 512)),
            full((512, 512)), full((1, 512)),
            full((512, 256)), full((1, 256)),
            full((256, 128)), full((1, 128)),
            full((128, 2)), full((1, 2)),
        ],
        out_specs=pl.BlockSpec((_BB, 2), lambda i: (i, 0)),
        out_shape=jax.ShapeDtypeStruct((nb * _BB, 2), jnp.float32),
    )(x, emb4, E, W1, b1, W2, b2, W3, b3, W4, b4, W5, b5)


_NP = 4
_BP = B // _NP


def kernel(x, table, W1, b1, W2, b2, W3, b3, W4, b4, W5, b5):
    E = jnp.repeat(jnp.eye(L, dtype=jnp.bfloat16), D, axis=1)
    bf16 = jnp.bfloat16
    Ws = (W1.astype(bf16), b1.reshape(1, -1),
          W2.astype(bf16), b2.reshape(1, -1),
          W3.astype(bf16), b3.reshape(1, -1),
          W4.astype(bf16), b4.reshape(1, -1),
          W5.astype(bf16), b5.reshape(1, -1))
    outs = []
    for p in range(_NP):
        xp = lax.slice_in_dim(x, p * _BP, (p + 1) * _BP, axis=0)
        idxP = xp.T.reshape(4, 16, _BP // 8, 8).transpose(2, 0, 3, 1).reshape(-1)
        emb4 = _sc_gather(table, idxP, _BP * L).reshape(_BP * L * D // 128, 128)
        outs.append(_mlp(xp, emb4, E, *Ws, nb=_BP // _BB))
    return jnp.concatenate(outs, axis=0)


# speedup vs baseline: 1.0343x; 1.0343x over previous
import functools

import jax
import jax.numpy as jnp
from jax import lax
from jax.experimental import pallas as pl
from jax.experimental.pallas import tpu as pltpu
from jax.experimental.pallas import tpu_sc as plsc

B, L, V, D = 16384, 64, 1000000, 8
BL = B * L
H = L * D
NSLAB = B // 8


_NC, _NS = 2, 16
_NW = _NC * _NS
_PER_W = BL // _NW
_CH = 2048
_NCH = _PER_W // _CH

def _sc_gather(table, idx, n_rows):
    mesh = plsc.VectorSubcoreMesh(core_axis_name="c", subcore_axis_name="s")
    per_w = n_rows // _NW
    nch = per_w // _CH

    @functools.partial(
        pl.kernel,
        mesh=mesh,
        compiler_params=pltpu.CompilerParams(use_tc_tiling_on_sc=False),
        out_type=jax.ShapeDtypeStruct((n_rows, D), jnp.float32),
        scratch_types=[
            pltpu.VMEM((_CH,), jnp.int32),
            pltpu.VMEM((_CH, D), jnp.float32),
            pltpu.VMEM((_CH,), jnp.int32),
            pltpu.VMEM((_CH, D), jnp.float32),
            pltpu.SemaphoreType.DMA,
            pltpu.SemaphoreType.DMA,
        ],
    )
    def gather_kernel(idx_hbm, table_hbm, out_hbm,
                      idx_a, rows_a, idx_b, rows_b, sem_a, sem_b):
        wid = lax.axis_index("s") * _NC + lax.axis_index("c")
        base = wid * per_w
        bufs = ((idx_a, rows_a, sem_a), (idx_b, rows_b, sem_b))

        pltpu.sync_copy(idx_hbm.at[pl.ds(base, _CH)], idx_a)
        cps = [pltpu.async_copy(table_hbm.at[idx_a], rows_a, sem_a), None]
        for i in range(nch):
            cur, nxt = i % 2, (i + 1) % 2
            if i + 1 < nch:
                iv, rv, sv = bufs[nxt]
                pltpu.sync_copy(
                    idx_hbm.at[pl.ds(base + (i + 1) * _CH, _CH)], iv)
                cps[nxt] = pltpu.async_copy(table_hbm.at[iv], rv, sv)
            _, rv, _ = bufs[cur]
            cps[cur].wait()
            pltpu.sync_copy(rv, out_hbm.at[pl.ds(base + i * _CH, _CH)])

    return gather_kernel(idx, table)


_BB = 512


def _mlp(x, emb4, E, W1, b1, W2, b2, W3, b3, W4, b4, W5, b5, nb):
    bf16 = jnp.bfloat16

    def dotb(a, w_ref):
        return jnp.dot(a.astype(bf16), w_ref[...],
                       preferred_element_type=jnp.float32)

    def mlp_kernel(x_ref, emb_ref, E_ref, W1_ref, b1_ref, W2_ref, b2_ref,
                   W3_ref, b3_ref, W4_ref, b4_ref, W5_ref, b5_ref, out_ref):
        m = (x_ref[...] != 0).astype(bf16)
        mask = jnp.dot(m, E_ref[...],
                       preferred_element_type=jnp.float32)
        e4 = emb_ref[...].reshape(_BB // 8, 4, 8, 128)
        h = None
        for ct in range(4):
            e = e4[:, ct, :, :].reshape(_BB, 128)
            e = e * mask[:, 128 * ct:128 * (ct + 1)]
            part = jnp.dot(e.astype(bf16),
                           W1_ref[pl.ds(128 * ct, 128), :],
                           preferred_element_type=jnp.float32)
            h = part if h is None else h + part
        h = jnp.maximum(h + b1_ref[...], 0.0)
        h = jnp.maximum(dotb(h, W2_ref) + b2_ref[...], 0.0)
        h = jnp.maximum(dotb(h, W3_ref) + b3_ref[...], 0.0)
        h = jnp.maximum(dotb(h, W4_ref) + b4_ref[...], 0.0)
        out_ref[...] = dotb(h, W5_ref) + b5_ref[...]

    grid = (nb,)
    full = lambda shape: pl.BlockSpec(shape, lambda i: tuple(0 for _ in shape))
    return pl.pallas_call(
        mlp_kernel,
        grid=grid,
        in_specs=[
            pl.BlockSpec((_BB, L), lambda i: (i, 0)),
            pl.BlockSpec((_BB * H // 128, 128), lambda i: (i, 0)),
            full((L, H)),
            full((512, 512)), full((1, 512)),
            full((512, 512)), full((1, 512)),
            full((512, 256)), full((1, 256)),
            full((256, 128)), full((1, 128)),
            full((128, 2)), full((1, 2)),
        ],
        out_specs=pl.BlockSpec((_BB, 2), lambda i: (i, 0)),
        out_shape=jax.ShapeDtypeStruct((nb * _BB, 2), jnp.float32),
    )(x, emb4, E, W1, b1, W2, b2, W3, b3, W4, b4, W5, b5)


_NP = 1
_BP = B // _NP


def kernel(x, table, W1, b1, W2, b2, W3, b3, W4, b4, W5, b5):
    E = jnp.repeat(jnp.eye(L, dtype=jnp.bfloat16), D, axis=1)
    bf16 = jnp.bfloat16
    Ws = (W1.astype(bf16), b1.reshape(1, -1),
          W2.astype(bf16), b2.reshape(1, -1),
          W3.astype(bf16), b3.reshape(1, -1),
          W4.astype(bf16), b4.reshape(1, -1),
          W5.astype(bf16), b5.reshape(1, -1))
    outs = []
    for p in range(_NP):
        xp = lax.slice_in_dim(x, p * _BP, (p + 1) * _BP, axis=0)
        idxP = xp.T.reshape(4, 16, _BP // 8, 8).transpose(2, 0, 3, 1).reshape(-1)
        emb4 = _sc_gather(table, idxP, _BP * L).reshape(_BP * L * D // 128, 128)
        outs.append(_mlp(xp, emb4, E, *Ws, nb=_BP // _BB))
    return jnp.concatenate(outs, axis=0)


# speedup vs baseline: 1.0359x; 1.0015x over previous
import functools

import jax
import jax.numpy as jnp
from jax import lax
from jax.experimental import pallas as pl
from jax.experimental.pallas import tpu as pltpu
from jax.experimental.pallas import tpu_sc as plsc

B, L, V, D = 16384, 64, 1000000, 8
BL = B * L
H = L * D
NSLAB = B // 8


_NC, _NS = 2, 16
_NW = _NC * _NS
_PER_W = BL // _NW
_CH = 2048
_NCH = _PER_W // _CH

def _sc_gather(table, idx, n_rows):
    mesh = plsc.VectorSubcoreMesh(core_axis_name="c", subcore_axis_name="s")
    per_w = n_rows // _NW
    nch = per_w // _CH

    @functools.partial(
        pl.kernel,
        mesh=mesh,
        compiler_params=pltpu.CompilerParams(use_tc_tiling_on_sc=False),
        out_type=jax.ShapeDtypeStruct((n_rows, D), jnp.float32),
        scratch_types=[
            pltpu.VMEM((_CH,), jnp.int32),
            pltpu.VMEM((_CH, D), jnp.float32),
            pltpu.VMEM((_CH,), jnp.int32),
            pltpu.VMEM((_CH, D), jnp.float32),
            pltpu.SemaphoreType.DMA,
            pltpu.SemaphoreType.DMA,
            pltpu.SemaphoreType.DMA,
            pltpu.SemaphoreType.DMA,
        ],
    )
    def gather_kernel(idx_hbm, table_hbm, out_hbm,
                      idx_a, rows_a, idx_b, rows_b,
                      sem_a, sem_b, wsem_a, wsem_b):
        wid = lax.axis_index("s") * _NC + lax.axis_index("c")
        base = wid * per_w
        bufs = ((idx_a, rows_a, sem_a, wsem_a),
                (idx_b, rows_b, sem_b, wsem_b))

        pltpu.sync_copy(idx_hbm.at[pl.ds(base, _CH)], idx_a)
        cps = [pltpu.async_copy(table_hbm.at[idx_a], rows_a, sem_a), None]
        wbs = [None, None]
        for i in range(nch):
            cur, nxt = i % 2, (i + 1) % 2
            if i + 1 < nch:
                iv, rv, sv, _ = bufs[nxt]
                pltpu.sync_copy(
                    idx_hbm.at[pl.ds(base + (i + 1) * _CH, _CH)], iv)
                if wbs[nxt] is not None:
                    wbs[nxt].wait()
                cps[nxt] = pltpu.async_copy(table_hbm.at[iv], rv, sv)
            _, rv, _, wv = bufs[cur]
            cps[cur].wait()
            wbs[cur] = pltpu.async_copy(
                rv, out_hbm.at[pl.ds(base + i * _CH, _CH)], wv)
        for wb in wbs:
            if wb is not None:
                wb.wait()

    return gather_kernel(idx, table)


_BB = 512


def _mlp(x, emb4, E, W1, b1, W2, b2, W3, b3, W4, b4, W5, b5, nb):
    bf16 = jnp.bfloat16

    def dotb(a, w_ref):
        return jnp.dot(a.astype(bf16), w_ref[...],
                       preferred_element_type=jnp.float32)

    def mlp_kernel(x_ref, emb_ref, E_ref, W1_ref, b1_ref, W2_ref, b2_ref,
                   W3_ref, b3_ref, W4_ref, b4_ref, W5_ref, b5_ref, out_ref):
        m = (x_ref[...] != 0).astype(bf16)
        mask = jnp.dot(m, E_ref[...],
                       preferred_element_type=jnp.float32)
        e4 = emb_ref[...].reshape(_BB // 8, 4, 8, 128)
        h = None
        for ct in range(4):
            e = e4[:, ct, :, :].reshape(_BB, 128)
            e = e * mask[:, 128 * ct:128 * (ct + 1)]
            part = jnp.dot(e.astype(bf16),
                           W1_ref[pl.ds(128 * ct, 128), :],
                           preferred_element_type=jnp.float32)
            h = part if h is None else h + part
        h = jnp.maximum(h + b1_ref[...], 0.0)
        h = jnp.maximum(dotb(h, W2_ref) + b2_ref[...], 0.0)
        h = jnp.maximum(dotb(h, W3_ref) + b3_ref[...], 0.0)
        h = jnp.maximum(dotb(h, W4_ref) + b4_ref[...], 0.0)
        out_ref[...] = dotb(h, W5_ref) + b5_ref[...]

    grid = (nb,)
    full = lambda shape: pl.BlockSpec(shape, lambda i: tuple(0 for _ in shape))
    return pl.pallas_call(
        mlp_kernel,
        grid=grid,
        in_specs=[
            pl.BlockSpec((_BB, L), lambda i: (i, 0)),
            pl.BlockSpec((_BB * H // 128, 128), lambda i: (i, 0)),
            full((L, H)),
            full((512, 512)), full((1, 512)),
            full((512, 512)), full((1, 512)),
            full((512, 256)), full((1, 256)),
            full((256, 128)), full((1, 128)),
            full((128, 2)), full((1, 2)),
        ],
        out_specs=pl.BlockSpec((_BB, 2), lambda i: (i, 0)),
        out_shape=jax.ShapeDtypeStruct((nb * _BB, 2), jnp.float32),
    )(x, emb4, E, W1, b1, W2, b2, W3, b3, W4, b4, W5, b5)


_NP = 1
_BP = B // _NP


def kernel(x, table, W1, b1, W2, b2, W3, b3, W4, b4, W5, b5):
    E = jnp.repeat(jnp.eye(L, dtype=jnp.bfloat16), D, axis=1)
    bf16 = jnp.bfloat16
    Ws = (W1.astype(bf16), b1.reshape(1, -1),
          W2.astype(bf16), b2.reshape(1, -1),
          W3.astype(bf16), b3.reshape(1, -1),
          W4.astype(bf16), b4.reshape(1, -1),
          W5.astype(bf16), b5.reshape(1, -1))
    outs = []
    for p in range(_NP):
        xp = lax.slice_in_dim(x, p * _BP, (p + 1) * _BP, axis=0)
        idxP = xp.T.reshape(4, 16, _BP // 8, 8).transpose(2, 0, 3, 1).reshape(-1)
        emb4 = _sc_gather(table, idxP, _BP * L).reshape(_BP * L * D // 128, 128)
        outs.append(_mlp(xp, emb4, E, *Ws, nb=_BP // _BB))
    return jnp.concatenate(outs, axis=0)


# speedup vs baseline: 1.0398x; 1.0037x over previous
import functools

import jax
import jax.numpy as jnp
from jax import lax
from jax.experimental import pallas as pl
from jax.experimental.pallas import tpu as pltpu
from jax.experimental.pallas import tpu_sc as plsc

B, L, V, D = 16384, 64, 1000000, 8
BL = B * L
H = L * D
NSLAB = B // 8


_NC, _NS = 2, 16
_NW = _NC * _NS
_PER_W = BL // _NW
_CH = 4096
_NCH = _PER_W // _CH

def _sc_gather(table, idx, n_rows):
    mesh = plsc.VectorSubcoreMesh(core_axis_name="c", subcore_axis_name="s")
    per_w = n_rows // _NW
    nch = per_w // _CH

    @functools.partial(
        pl.kernel,
        mesh=mesh,
        compiler_params=pltpu.CompilerParams(use_tc_tiling_on_sc=False),
        out_type=jax.ShapeDtypeStruct((n_rows, D), jnp.float32),
        scratch_types=[
            pltpu.VMEM((_CH,), jnp.int32),
            pltpu.VMEM((_CH, D), jnp.float32),
            pltpu.VMEM((_CH,), jnp.int32),
            pltpu.VMEM((_CH, D), jnp.float32),
            pltpu.SemaphoreType.DMA,
            pltpu.SemaphoreType.DMA,
            pltpu.SemaphoreType.DMA,
            pltpu.SemaphoreType.DMA,
        ],
    )
    def gather_kernel(idx_hbm, table_hbm, out_hbm,
                      idx_a, rows_a, idx_b, rows_b,
                      sem_a, sem_b, wsem_a, wsem_b):
        wid = lax.axis_index("s") * _NC + lax.axis_index("c")
        base = wid * per_w
        bufs = ((idx_a, rows_a, sem_a, wsem_a),
                (idx_b, rows_b, sem_b, wsem_b))

        pltpu.sync_copy(idx_hbm.at[pl.ds(base, _CH)], idx_a)
        cps = [pltpu.async_copy(table_hbm.at[idx_a], rows_a, sem_a), None]
        wbs = [None, None]
        for i in range(nch):
            cur, nxt = i % 2, (i + 1) % 2
            if i + 1 < nch:
                iv, rv, sv, _ = bufs[nxt]
                pltpu.sync_copy(
                    idx_hbm.at[pl.ds(base + (i + 1) * _CH, _CH)], iv)
                if wbs[nxt] is not None:
                    wbs[nxt].wait()
                cps[nxt] = pltpu.async_copy(table_hbm.at[iv], rv, sv)
            _, rv, _, wv = bufs[cur]
            cps[cur].wait()
            wbs[cur] = pltpu.async_copy(
                rv, out_hbm.at[pl.ds(base + i * _CH, _CH)], wv)
        for wb in wbs:
            if wb is not None:
                wb.wait()

    return gather_kernel(idx, table)


_BB = 512


def _mlp(x, emb4, E, W1, b1, W2, b2, W3, b3, W4, b4, W5, b5, nb):
    bf16 = jnp.bfloat16

    def dotb(a, w_ref):
        return jnp.dot(a.astype(bf16), w_ref[...],
                       preferred_element_type=jnp.float32)

    def mlp_kernel(x_ref, emb_ref, E_ref, W1_ref, b1_ref, W2_ref, b2_ref,
                   W3_ref, b3_ref, W4_ref, b4_ref, W5_ref, b5_ref, out_ref):
        m = (x_ref[...] != 0).astype(bf16)
        mask = jnp.dot(m, E_ref[...],
                       preferred_element_type=jnp.float32)
        e4 = emb_ref[...].reshape(_BB // 8, 4, 8, 128)
        h = None
        for ct in range(4):
            e = e4[:, ct, :, :].reshape(_BB, 128)
            e = e * mask[:, 128 * ct:128 * (ct + 1)]
            part = jnp.dot(e.astype(bf16),
                           W1_ref[pl.ds(128 * ct, 128), :],
                           preferred_element_type=jnp.float32)
            h = part if h is None else h + part
        h = jnp.maximum(h + b1_ref[...], 0.0)
        h = jnp.maximum(dotb(h, W2_ref) + b2_ref[...], 0.0)
        h = jnp.maximum(dotb(h, W3_ref) + b3_ref[...], 0.0)
        h = jnp.maximum(dotb(h, W4_ref) + b4_ref[...], 0.0)
        out_ref[...] = dotb(h, W5_ref) + b5_ref[...]

    grid = (nb,)
    full = lambda shape: pl.BlockSpec(shape, lambda i: tuple(0 for _ in shape))
    return pl.pallas_call(
        mlp_kernel,
        grid=grid,
        in_specs=[
            pl.BlockSpec((_BB, L), lambda i: (i, 0)),
            pl.BlockSpec((_BB * H // 128, 128), lambda i: (i, 0)),
            full((L, H)),
            full((512, 512)), full((1, 512)),
            full((512, 512)), full((1, 512)),
            full((512, 256)), full((1, 256)),
            full((256, 128)), full((1, 128)),
            full((128, 2)), full((1, 2)),
        ],
        out_specs=pl.BlockSpec((_BB, 2), lambda i: (i, 0)),
        out_shape=jax.ShapeDtypeStruct((nb * _BB, 2), jnp.float32),
    )(x, emb4, E, W1, b1, W2, b2, W3, b3, W4, b4, W5, b5)


_NP = 1
_BP = B // _NP


def kernel(x, table, W1, b1, W2, b2, W3, b3, W4, b4, W5, b5):
    E = jnp.repeat(jnp.eye(L, dtype=jnp.bfloat16), D, axis=1)
    bf16 = jnp.bfloat16
    Ws = (W1.astype(bf16), b1.reshape(1, -1),
          W2.astype(bf16), b2.reshape(1, -1),
          W3.astype(bf16), b3.reshape(1, -1),
          W4.astype(bf16), b4.reshape(1, -1),
          W5.astype(bf16), b5.reshape(1, -1))
    outs = []
    for p in range(_NP):
        xp = lax.slice_in_dim(x, p * _BP, (p + 1) * _BP, axis=0)
        idxP = xp.T.reshape(4, 16, _BP // 8, 8).transpose(2, 0, 3, 1).reshape(-1)
        emb4 = _sc_gather(table, idxP, _BP * L).reshape(_BP * L * D // 128, 128)
        outs.append(_mlp(xp, emb4, E, *Ws, nb=_BP // _BB))
    return jnp.concatenate(outs, axis=0)
